# Initial kernel scaffold; baseline (speedup 1.0000x reference)
import jax
import jax.numpy as jnp
from jax.experimental import pallas as pl


def kernel(node_embs, leave_inds, W1, b1, W2, b2):
    raise NotImplementedError("write your pallas kernel here")


# speedup vs baseline: 2.1588x; 2.1588x over previous
import functools

import jax
import jax.numpy as jnp
from jax import lax
from jax.experimental import pallas as pl
from jax.experimental.pallas import tpu as pltpu
from jax.experimental.pallas import tpu_sc as plsc

_N_NODES = 100000
_D = 128
_N_LEAVES = 5000
_LPAD = 5120


def _make_sc_gather():
    info = plsc.get_sparse_core_info()
    nc, ns = info.num_cores, info.num_subcores
    nw = nc * ns
    b_per_w = _LPAD // nw
    mesh = plsc.VectorSubcoreMesh(core_axis_name="c", subcore_axis_name="s")

    @functools.partial(
        pl.kernel,
        mesh=mesh,
        out_type=jax.ShapeDtypeStruct((_LPAD, _D), jnp.float32),
        scratch_types=[
            pltpu.VMEM((b_per_w,), jnp.int32),
            pltpu.VMEM((b_per_w, _D), jnp.float32),
            pltpu.SemaphoreType.DMA,
        ],
    )
    def gather_k(table_hbm, idx_hbm, out_hbm, idx_v, rows_v, sem):
        wid = lax.axis_index("s") * nc + lax.axis_index("c")
        base = wid * b_per_w
        pltpu.sync_copy(idx_hbm.at[pl.ds(base, b_per_w)], idx_v)
        pltpu.async_copy(table_hbm.at[idx_v], rows_v, sem).wait()
        pltpu.sync_copy(rows_v, out_hbm.at[pl.ds(base, b_per_w)])

    return gather_k


_sc_gather_cache = []


def _sc_gather(table, idx):
    if not _sc_gather_cache:
        _sc_gather_cache.append(_make_sc_gather())
    return _sc_gather_cache[0](table, idx)


_SUM_BLOCK = 4000


def _sum_body(x_ref, out_ref):
    @pl.when(pl.program_id(0) == 0)
    def _():
        out_ref[...] = jnp.zeros_like(out_ref)

    out_ref[...] += jnp.sum(x_ref[...], axis=0, keepdims=True)


def _col_sum(node_embs):
    return pl.pallas_call(
        _sum_body,
        grid=(_N_NODES // _SUM_BLOCK,),
        in_specs=[pl.BlockSpec((_SUM_BLOCK, _D), lambda i: (i, 0))],
        out_specs=pl.BlockSpec((1, _D), lambda i: (0, 0)),
        out_shape=jax.ShapeDtypeStruct((1, _D), jnp.float32),
    )(node_embs)


def _mlp_body(g_ref, sum_ref, w1_ref, b1_ref, w2_ref, b2_ref, out_ref):
    graph = sum_ref[...] * (1.0 / _N_NODES)
    w1 = w1_ref[...]
    c = jnp.dot(graph, w1[_D:, :], preferred_element_type=jnp.float32)
    c = c + b1_ref[...]
    h = jnp.dot(g_ref[...], w1[:_D, :], preferred_element_type=jnp.float32)
    h = jnp.maximum(h + c, 0.0)
    s = jnp.sum(h * w2_ref[...], axis=1, keepdims=True) + b2_ref[0, 0]
    row = lax.broadcasted_iota(jnp.int32, (_LPAD, 1), 0)
    s = jnp.where(row < _N_LEAVES, s, -jnp.inf)
    e = jnp.exp(s - jnp.max(s))
    out_ref[...] = e / jnp.sum(e)


def _mlp(g, col_sum, W1, b1, w2_row, b2):
    return pl.pallas_call(
        _mlp_body,
        in_specs=[
            pl.BlockSpec((_LPAD, _D), lambda: (0, 0)),
            pl.BlockSpec((1, _D), lambda: (0, 0)),
            pl.BlockSpec((2 * _D, _D), lambda: (0, 0)),
            pl.BlockSpec((1, _D), lambda: (0, 0)),
            pl.BlockSpec((1, _D), lambda: (0, 0)),
            pl.BlockSpec((1, 1), lambda: (0, 0)),
        ],
        out_specs=pl.BlockSpec((_LPAD, 1), lambda: (0, 0)),
        out_shape=jax.ShapeDtypeStruct((_LPAD, 1), jnp.float32),
    )(g, col_sum, W1, b1, w2_row, b2)


def kernel(node_embs, leave_inds, W1, b1, W2, b2):
    idx = jnp.zeros((_LPAD,), jnp.int32).at[:_N_LEAVES].set(
        leave_inds.astype(jnp.int32))
    g = _sc_gather(node_embs, idx)
    col_sum = _col_sum(node_embs)
    scores = _mlp(g, col_sum, W1, b1.reshape(1, _D), W2.reshape(1, _D),
                  b2.reshape(1, 1))
    return scores[:_N_LEAVES, 0]


# speedup vs baseline: 2.3821x; 1.1034x over previous
import functools

import jax
import jax.numpy as jnp
from jax import lax
from jax.experimental import pallas as pl
from jax.experimental.pallas import tpu as pltpu
from jax.experimental.pallas import tpu_sc as plsc

_N_NODES = 100000
_D = 128
_N_LEAVES = 5000
_LPAD = 5120


def _make_sc_gather():
    info = plsc.get_sparse_core_info()
    nc, ns = info.num_cores, info.num_subcores
    nw = nc * ns
    b_per_w = _LPAD // nw
    mesh = plsc.VectorSubcoreMesh(core_axis_name="c", subcore_axis_name="s")

    @functools.partial(
        pl.kernel,
        mesh=mesh,
        out_type=jax.ShapeDtypeStruct((_LPAD, _D), jnp.float32),
        scratch_types=[
            pltpu.VMEM((b_per_w,), jnp.int32),
            pltpu.VMEM((b_per_w, _D), jnp.float32),
            pltpu.SemaphoreType.DMA,
        ],
    )
    def gather_k(table_hbm, idx_hbm, out_hbm, idx_v, rows_v, sem):
        wid = lax.axis_index("s") * nc + lax.axis_index("c")
        base = wid * b_per_w
        pltpu.sync_copy(idx_hbm.at[pl.ds(base, b_per_w)], idx_v)
        pltpu.async_copy(table_hbm.at[idx_v], rows_v, sem).wait()
        pltpu.sync_copy(rows_v, out_hbm.at[pl.ds(base, b_per_w)])

    return gather_k


_sc_gather_cache = []


def _sc_gather(table, idx):
    if not _sc_gather_cache:
        _sc_gather_cache.append(_make_sc_gather())
    return _sc_gather_cache[0](table, idx)


_SUM_BLOCK = 10000


def _sum_body(x_ref, out_ref):
    @pl.when(pl.program_id(0) == 0)
    def _():
        out_ref[...] = jnp.zeros_like(out_ref)

    out_ref[...] += jnp.sum(x_ref[...], axis=0, keepdims=True)


def _col_sum(node_embs):
    return pl.pallas_call(
        _sum_body,
        grid=(_N_NODES // _SUM_BLOCK,),
        in_specs=[pl.BlockSpec((_SUM_BLOCK, _D), lambda i: (i, 0))],
        out_specs=pl.BlockSpec((1, _D), lambda i: (0, 0)),
        out_shape=jax.ShapeDtypeStruct((1, _D), jnp.float32),
    )(node_embs)


def _mlp_body(g_ref, sum_ref, w1_ref, b1_ref, w2_ref, b2_ref, out_ref):
    graph = sum_ref[...] * (1.0 / _N_NODES)
    w1 = w1_ref[...]
    c = jnp.dot(graph, w1[_D:, :], preferred_element_type=jnp.float32)
    c = c + b1_ref[...]
    g = g_ref[...][:_N_LEAVES, :]
    h = jnp.dot(g, w1[:_D, :], preferred_element_type=jnp.float32)
    h = jnp.maximum(h + c, 0.0)
    s = jnp.sum(h * w2_ref[...], axis=1, keepdims=True) + b2_ref[0, 0]
    e = jnp.exp(s - jnp.max(s))
    out_ref[...] = e / jnp.sum(e)


def _mlp(g, col_sum, W1, b1, w2_row, b2):
    return pl.pallas_call(
        _mlp_body,
        in_specs=[
            pl.BlockSpec((_LPAD, _D), lambda: (0, 0)),
            pl.BlockSpec((1, _D), lambda: (0, 0)),
            pl.BlockSpec((2 * _D, _D), lambda: (0, 0)),
            pl.BlockSpec((1, _D), lambda: (0, 0)),
            pl.BlockSpec((1, _D), lambda: (0, 0)),
            pl.BlockSpec((1, 1), lambda: (0, 0)),
        ],
        out_specs=pl.BlockSpec((_N_LEAVES, 1), lambda: (0, 0)),
        out_shape=jax.ShapeDtypeStruct((_N_LEAVES, 1), jnp.float32),
    )(g, col_sum, W1, b1, w2_row, b2)


def kernel(node_embs, leave_inds, W1, b1, W2, b2):
    idx = jnp.zeros((_LPAD,), jnp.int32).at[:_N_LEAVES].set(
        leave_inds.astype(jnp.int32))
    g = _sc_gather(node_embs, idx)
    col_sum = _col_sum(node_embs)
    scores = _mlp(g, col_sum, W1, b1.reshape(1, _D), W2.reshape(1, _D),
                  b2.reshape(1, 1))
    return scores.reshape(_N_LEAVES)


# speedup vs baseline: 2.6713x; 1.1214x over previous
import functools

import jax
import jax.numpy as jnp
from jax import lax
from jax.experimental import pallas as pl
from jax.experimental.pallas import tpu as pltpu
from jax.experimental.pallas import tpu_sc as plsc

_N_NODES = 100000
_D = 128
_N_LEAVES = 5000
_LPAD = 5120


def _make_sc_gather():
    info = plsc.get_sparse_core_info()
    nc, ns = info.num_cores, info.num_subcores
    nw = nc * ns
    b_per_w = _LPAD // nw
    mesh = plsc.VectorSubcoreMesh(core_axis_name="c", subcore_axis_name="s")
    b_tail = _N_LEAVES - (nw - 1) * b_per_w

    @functools.partial(
        pl.kernel,
        mesh=mesh,
        out_type=jax.ShapeDtypeStruct((_LPAD, _D), jnp.float32),
        scratch_types=[
            pltpu.VMEM((b_per_w,), jnp.int32),
            pltpu.VMEM((b_per_w, _D), jnp.float32),
            pltpu.SemaphoreType.DMA,
        ],
    )
    def gather_k(table_hbm, idx_hbm, out_hbm, idx_v, rows_v, sem):
        wid = lax.axis_index("s") * nc + lax.axis_index("c")
        base = wid * b_per_w

        @pl.when(wid < nw - 1)
        def _full():
            pltpu.sync_copy(idx_hbm.at[pl.ds(base, b_per_w)], idx_v)
            pltpu.async_copy(table_hbm.at[idx_v], rows_v, sem).wait()
            pltpu.sync_copy(rows_v, out_hbm.at[pl.ds(base, b_per_w)])

        @pl.when(wid == nw - 1)
        def _tail():
            pltpu.sync_copy(idx_hbm.at[pl.ds(base, b_tail)],
                            idx_v.at[pl.ds(0, b_tail)])
            pltpu.async_copy(table_hbm.at[idx_v.at[pl.ds(0, b_tail)]],
                             rows_v.at[pl.ds(0, b_tail)], sem).wait()
            pltpu.sync_copy(rows_v.at[pl.ds(0, b_tail)],
                            out_hbm.at[pl.ds(base, b_tail)])

    return gather_k


_sc_gather_cache = []


def _sc_gather(table, idx):
    if not _sc_gather_cache:
        _sc_gather_cache.append(_make_sc_gather())
    return _sc_gather_cache[0](table, idx)


_SUM_BLOCK = 10000


def _sum_body(x_ref, out_ref):
    @pl.when(pl.program_id(0) == 0)
    def _():
        out_ref[...] = jnp.zeros_like(out_ref)

    out_ref[...] += jnp.sum(x_ref[...], axis=0, keepdims=True)


def _col_sum(node_embs):
    return pl.pallas_call(
        _sum_body,
        grid=(_N_NODES // _SUM_BLOCK,),
        in_specs=[pl.BlockSpec((_SUM_BLOCK, _D), lambda i: (i, 0))],
        out_specs=pl.BlockSpec((1, _D), lambda i: (0, 0)),
        out_shape=jax.ShapeDtypeStruct((1, _D), jnp.float32),
    )(node_embs)


def _mlp_body(g_ref, sum_ref, w1_ref, b1_ref, w2_ref, b2_ref, out_ref):
    graph = sum_ref[...] * (1.0 / _N_NODES)
    w1 = w1_ref[...]
    c = jnp.dot(graph, w1[_D:, :], preferred_element_type=jnp.float32)
    c = c + b1_ref[...]
    g = g_ref[...][:_N_LEAVES, :]
    h = jnp.dot(g, w1[:_D, :], preferred_element_type=jnp.float32)
    h = jnp.maximum(h + c, 0.0)
    s = jnp.sum(h * w2_ref[...], axis=1, keepdims=True) + b2_ref[0, 0]
    e = jnp.exp(s - jnp.max(s))
    out_ref[...] = e / jnp.sum(e)


def _mlp(g, col_sum, W1, b1, w2_row, b2):
    return pl.pallas_call(
        _mlp_body,
        in_specs=[
            pl.BlockSpec((_LPAD, _D), lambda: (0, 0)),
            pl.BlockSpec((1, _D), lambda: (0, 0)),
            pl.BlockSpec((2 * _D, _D), lambda: (0, 0)),
            pl.BlockSpec((1, _D), lambda: (0, 0)),
            pl.BlockSpec((1, _D), lambda: (0, 0)),
            pl.BlockSpec((1, 1), lambda: (0, 0)),
        ],
        out_specs=pl.BlockSpec((_N_LEAVES, 1), lambda: (0, 0)),
        out_shape=jax.ShapeDtypeStruct((_N_LEAVES, 1), jnp.float32),
    )(g, col_sum, W1, b1, w2_row, b2)


def kernel(node_embs, leave_inds, W1, b1, W2, b2):
    g = _sc_gather(node_embs, leave_inds.astype(jnp.int32))
    col_sum = _col_sum(node_embs)
    scores = _mlp(g, col_sum, W1, b1.reshape(1, _D), W2.reshape(1, _D),
                  b2.reshape(1, 1))
    return scores.reshape(_N_LEAVES)


# speedup vs baseline: 2.8246x; 1.0574x over previous
import functools

import jax
import jax.numpy as jnp
from jax import lax
from jax.experimental import pallas as pl
from jax.experimental.pallas import tpu as pltpu
from jax.experimental.pallas import tpu_sc as plsc

_N_NODES = 100000
_D = 128
_N_LEAVES = 5000
_LPAD = 5120


def _make_sc_gather():
    info = plsc.get_sparse_core_info()
    nc, ns = info.num_cores, info.num_subcores
    nw = nc * ns
    b_per_w = _LPAD // nw
    mesh = plsc.VectorSubcoreMesh(core_axis_name="c", subcore_axis_name="s")
    b_tail = _N_LEAVES - (nw - 1) * b_per_w

    @functools.partial(
        pl.kernel,
        mesh=mesh,
        out_type=jax.ShapeDtypeStruct((_LPAD, _D), jnp.float32),
        scratch_types=[
            pltpu.VMEM((b_per_w,), jnp.int32),
            pltpu.VMEM((b_per_w, _D), jnp.float32),
            pltpu.SemaphoreType.DMA,
        ],
    )
    def gather_k(table_hbm, idx_hbm, out_hbm, idx_v, rows_v, sem):
        wid = lax.axis_index("s") * nc + lax.axis_index("c")
        base = wid * b_per_w

        @pl.when(wid < nw - 1)
        def _full():
            pltpu.sync_copy(idx_hbm.at[pl.ds(base, b_per_w)], idx_v)
            pltpu.async_copy(table_hbm.at[idx_v], rows_v, sem).wait()
            pltpu.sync_copy(rows_v, out_hbm.at[pl.ds(base, b_per_w)])

        @pl.when(wid == nw - 1)
        def _tail():
            pltpu.sync_copy(idx_hbm.at[pl.ds(base, b_tail)],
                            idx_v.at[pl.ds(0, b_tail)])
            pltpu.async_copy(table_hbm.at[idx_v.at[pl.ds(0, b_tail)]],
                             rows_v.at[pl.ds(0, b_tail)], sem).wait()
            pltpu.sync_copy(rows_v.at[pl.ds(0, b_tail)],
                            out_hbm.at[pl.ds(base, b_tail)])

    return gather_k


_sc_gather_cache = []


def _sc_gather(table, idx):
    if not _sc_gather_cache:
        _sc_gather_cache.append(_make_sc_gather())
    return _sc_gather_cache[0](table, idx)


_SUM_BLOCK = 10000


def _sum_body(x_ref, out_ref):
    @pl.when(pl.program_id(0) == 0)
    def _():
        out_ref[...] = jnp.zeros_like(out_ref)

    out_ref[...] += jnp.sum(x_ref[...], axis=0, keepdims=True)


def _col_sum(node_embs):
    return pl.pallas_call(
        _sum_body,
        grid=(_N_NODES // _SUM_BLOCK,),
        in_specs=[pl.BlockSpec((_SUM_BLOCK, _D), lambda i: (i, 0))],
        out_specs=pl.BlockSpec((1, _D), lambda i: (0, 0)),
        out_shape=jax.ShapeDtypeStruct((1, _D), jnp.float32),
    )(node_embs)


def _mlp_body(g_ref, sum_ref, w1_ref, b1_ref, w2_ref, b2_ref, out_ref):
    graph = sum_ref[...] * (1.0 / _N_NODES)
    w1 = w1_ref[...]
    c = jnp.dot(graph, w1[_D:, :], preferred_element_type=jnp.float32)
    c = c + b1_ref[...]
    g = g_ref[...][:_N_LEAVES, :]
    h = jnp.dot(g, w1[:_D, :], preferred_element_type=jnp.float32)
    h = jnp.maximum(h + c, 0.0)
    s = jnp.sum(h * w2_ref[...], axis=1, keepdims=True) + b2_ref[0, 0]
    e = jnp.exp(s - jnp.max(s))
    out_ref[...] = (e / jnp.sum(e)).reshape(_N_LEAVES)


def _mlp(g, col_sum, W1, b1, w2_row, b2):
    return pl.pallas_call(
        _mlp_body,
        in_specs=[
            pl.BlockSpec((_LPAD, _D), lambda: (0, 0)),
            pl.BlockSpec((1, _D), lambda: (0, 0)),
            pl.BlockSpec((2 * _D, _D), lambda: (0, 0)),
            pl.BlockSpec((1, _D), lambda: (0, 0)),
            pl.BlockSpec((1, _D), lambda: (0, 0)),
            pl.BlockSpec((1, 1), lambda: (0, 0)),
        ],
        out_specs=pl.BlockSpec((_N_LEAVES,), lambda: (0,)),
        out_shape=jax.ShapeDtypeStruct((_N_LEAVES,), jnp.float32),
    )(g, col_sum, W1, b1, w2_row, b2)


def kernel(node_embs, leave_inds, W1, b1, W2, b2):
    g = _sc_gather(node_embs, leave_inds.astype(jnp.int32))
    col_sum = _col_sum(node_embs)
    return _mlp(g, col_sum, W1, b1.reshape(1, _D), W2.reshape(1, _D),
                b2.reshape(1, 1))


# speedup vs baseline: 2.8800x; 1.0196x over previous
import functools

import jax
import jax.numpy as jnp
from jax import lax
from jax.experimental import pallas as pl
from jax.experimental.pallas import tpu as pltpu
from jax.experimental.pallas import tpu_sc as plsc

_N_NODES = 100000
_D = 128
_N_LEAVES = 5000
_LPAD = 5120


def _make_sc_gather():
    info = plsc.get_sparse_core_info()
    nc, ns = info.num_cores, info.num_subcores
    nw = nc * ns
    b_per_w = _LPAD // nw
    mesh = plsc.VectorSubcoreMesh(core_axis_name="c", subcore_axis_name="s")
    b_tail = _N_LEAVES - (nw - 1) * b_per_w

    @functools.partial(
        pl.kernel,
        mesh=mesh,
        out_type=jax.ShapeDtypeStruct((_LPAD, _D), jnp.float32),
        scratch_types=[
            pltpu.VMEM((b_per_w,), jnp.int32),
            pltpu.VMEM((b_per_w, _D), jnp.float32),
            pltpu.SemaphoreType.DMA,
        ],
    )
    def gather_k(table_hbm, idx_hbm, out_hbm, idx_v, rows_v, sem):
        wid = lax.axis_index("s") * nc + lax.axis_index("c")
        base = wid * b_per_w

        @pl.when(wid < nw - 1)
        def _full():
            pltpu.sync_copy(idx_hbm.at[pl.ds(base, b_per_w)], idx_v)
            pltpu.async_copy(table_hbm.at[idx_v], rows_v, sem).wait()
            pltpu.sync_copy(rows_v, out_hbm.at[pl.ds(base, b_per_w)])

        @pl.when(wid == nw - 1)
        def _tail():
            pltpu.sync_copy(idx_hbm.at[pl.ds(base, b_tail)],
                            idx_v.at[pl.ds(0, b_tail)])
            pltpu.async_copy(table_hbm.at[idx_v.at[pl.ds(0, b_tail)]],
                             rows_v.at[pl.ds(0, b_tail)], sem).wait()
            pltpu.sync_copy(rows_v.at[pl.ds(0, b_tail)],
                            out_hbm.at[pl.ds(base, b_tail)])

    return gather_k


_sc_gather_cache = []


def _sc_gather(table, idx):
    if not _sc_gather_cache:
        _sc_gather_cache.append(_make_sc_gather())
    return _sc_gather_cache[0](table, idx)


_SUM_BLOCK = 20000


def _sum_body(x_ref, out_ref):
    @pl.when(pl.program_id(0) == 0)
    def _():
        out_ref[...] = jnp.zeros_like(out_ref)

    out_ref[...] += jnp.sum(x_ref[...], axis=0, keepdims=True)


def _col_sum(node_embs):
    return pl.pallas_call(
        _sum_body,
        grid=(_N_NODES // _SUM_BLOCK,),
        in_specs=[pl.BlockSpec((_SUM_BLOCK, _D), lambda i: (i, 0))],
        out_specs=pl.BlockSpec((1, _D), lambda i: (0, 0)),
        out_shape=jax.ShapeDtypeStruct((1, _D), jnp.float32),
    )(node_embs)


def _mlp_body(g_ref, sum_ref, w1_ref, b1_ref, w2_ref, b2_ref, out_ref):
    graph = sum_ref[...] * (1.0 / _N_NODES)
    w1 = w1_ref[...]
    c = jnp.dot(graph, w1[_D:, :], preferred_element_type=jnp.float32)
    c = c + b1_ref[...]
    g = g_ref[...][:_N_LEAVES, :]
    h = jnp.dot(g, w1[:_D, :], preferred_element_type=jnp.float32)
    h = jnp.maximum(h + c, 0.0)
    s = jnp.sum(h * w2_ref[...], axis=1, keepdims=True) + b2_ref[0, 0]
    e = jnp.exp(s - jnp.max(s))
    out_ref[...] = (e / jnp.sum(e)).reshape(_N_LEAVES)


def _mlp(g, col_sum, W1, b1, w2_row, b2):
    return pl.pallas_call(
        _mlp_body,
        in_specs=[
            pl.BlockSpec((_LPAD, _D), lambda: (0, 0)),
            pl.BlockSpec((1, _D), lambda: (0, 0)),
            pl.BlockSpec((2 * _D, _D), lambda: (0, 0)),
            pl.BlockSpec((1, _D), lambda: (0, 0)),
            pl.BlockSpec((1, _D), lambda: (0, 0)),
            pl.BlockSpec((1, 1), lambda: (0, 0)),
        ],
        out_specs=pl.BlockSpec((_N_LEAVES,), lambda: (0,)),
        out_shape=jax.ShapeDtypeStruct((_N_LEAVES,), jnp.float32),
    )(g, col_sum, W1, b1, w2_row, b2)


def kernel(node_embs, leave_inds, W1, b1, W2, b2):
    g = _sc_gather(node_embs, leave_inds.astype(jnp.int32))
    col_sum = _col_sum(node_embs)
    return _mlp(g, col_sum, W1, b1.reshape(1, _D), W2.reshape(1, _D),
                b2.reshape(1, 1))
